# baseline (speedup 1.0000x reference)
import jax
import jax.numpy as jnp
from jax.experimental import pallas as pl
from jax.experimental.pallas import tpu as pltpu

DIM = 768
D_STATE = 16
D_CONV = 4
E = 8
D_INNER = 2 * DIM
DT_RANK = (DIM + 15) // 16
HID = 4 * DIM
L = 2048
SB = 256
NSB = L // SB
KS = 16

_f32 = jnp.float32
_bf16 = jnp.bfloat16


import functools
from jax import lax
from jax.experimental.pallas import tpu_sc as plsc

NBLK_MOE = 24
MAX_P = NBLK_MOE * SB
NW = 32
_i32 = jnp.int32

_mesh = plsc.VectorSubcoreMesh(core_axis_name="c", subcore_axis_name="s")

def _widx():
    return lax.axis_index("s") * 2 + lax.axis_index("c")


@functools.partial(
    pl.kernel,
    mesh=_mesh,
    out_type=jax.ShapeDtypeStruct((E * 16,), _i32),
    scratch_types=[
        pltpu.VMEM((L,), _i32),
        pltpu.VMEM((L,), _i32),
        pltpu.VMEM((E * 16,), _i32),
    ],
)
def sc_hist(eA_hbm, eB_hbm, cnt_hbm, eA_v, eB_v, cnt_v):
    wid = _widx()

    @pl.when(wid == 0)
    def _():
        pltpu.sync_copy(eA_hbm, eA_v)
        pltpu.sync_copy(eB_hbm, eB_v)

        def hbody(k, cnt):
            va = eA_v[pl.ds(k * 16, 16)]
            vb = eB_v[pl.ds(k * 16, 16)]
            return tuple(cnt[e]
                         + jnp.where(va == e, 1, 0)
                         + jnp.where(vb == e, 1, 0)
                         for e in range(E))
        cnt = lax.fori_loop(0, L // 16, hbody,
                            (jnp.zeros((16,), _i32),) * E)
        for e in range(E):
            cnt_v[pl.ds(e * 16, 16)] = cnt[e]
        pltpu.sync_copy(cnt_v, cnt_hbm)


def _tc_meta_body(cnt_ref, base_ref, blkm_ref):
    cnt = cnt_ref[...].astype(_f32)
    tot = jnp.sum(cnt, axis=1, keepdims=True)
    nb = jnp.floor((tot + SB - 1) / SB)
    r = jax.lax.broadcasted_iota(jnp.int32, (E, E), 0)
    c = jax.lax.broadcasted_iota(jnp.int32, (E, E), 1)
    triE = (r > c).astype(_bf16)
    offs = SB * jax.lax.dot_general(triE, nb.astype(_bf16),
                                    (((1,), (0,)), ((), ())),
                                    preferred_element_type=_f32)
    bs = offs / SB
    rl = jax.lax.broadcasted_iota(jnp.int32, (16, 16), 0)
    cl = jax.lax.broadcasted_iota(jnp.int32, (16, 16), 1)
    triL = (rl < cl).astype(_bf16)
    lpre = jax.lax.dot_general(cnt.astype(_bf16), triL,
                               (((1,), (0,)), ((), ())),
                               preferred_element_type=_f32)
    base_ref[...] = (offs + lpre).astype(_i32)
    bv = jax.lax.broadcasted_iota(jnp.int32, (1, 48), 1).astype(_f32)
    acc = jnp.zeros((1, 48), _f32)
    for e in range(1, E):
        acc = acc + jnp.where(bv >= bs[e, 0], 1.0, 0.0)
    nblk = bs[E - 1, 0] + nb[E - 1, 0]
    out = jnp.where(bv < 32.0, acc, nblk)
    blkm_ref[...] = out.astype(_i32)


def tc_meta(cnt):
    return pl.pallas_call(
        _tc_meta_body,
        out_shape=[
            jax.ShapeDtypeStruct((E, 16), _i32),
            jax.ShapeDtypeStruct((1, 48), _i32),
        ],
    )(cnt)


@functools.partial(
    pl.kernel,
    mesh=_mesh,
    out_type=[
        jax.ShapeDtypeStruct((L,), _i32),
        jax.ShapeDtypeStruct((L,), _i32),
        jax.ShapeDtypeStruct((MAX_P,), _i32),
    ],
    scratch_types=[
        pltpu.VMEM((L,), _i32),
        pltpu.VMEM((L,), _i32),
        pltpu.VMEM((E * 16,), _i32),
        pltpu.VMEM((L,), _i32),
        pltpu.VMEM((L,), _i32),
        pltpu.VMEM((L,), _i32),
        pltpu.VMEM((MAX_P,), _i32),
        pltpu.SemaphoreType.DMA,
    ],
)
def sc_pos(eA_hbm, eB_hbm, base_hbm, posA_hbm, posB_hbm, tok_hbm,
           eA_v, eB_v, base_v, posA_v, posB_v, tid_v, zero_v, sem):
    wid = _widx()

    @pl.when(wid == 0)
    def _():
        pltpu.sync_copy(eA_hbm, eA_v)
        pltpu.sync_copy(eB_hbm, eB_v)
        pltpu.sync_copy(base_hbm, base_v)
        lanes = lax.iota(_i32, 16)
        zeros16 = jnp.zeros((16,), _i32)

        def zbody(k, carry):
            zero_v[pl.ds(k * 16, 16)] = zeros16
            return carry
        lax.fori_loop(0, MAX_P // 16, zbody, 0)
        pltpu.sync_copy(zero_v, tok_hbm)

        def tbody(k, carry):
            tid_v[pl.ds(k * 16, 16)] = lanes + k * 16
            return carry
        lax.fori_loop(0, L // 16, tbody, 0)

        run0 = tuple(base_v[pl.ds(e * 16, 16)] for e in range(E))

        def pbody(k, run):
            va = eA_v[pl.ds(k * 16, 16)]
            pos = jnp.zeros((16,), _i32)
            for e in range(E):
                pos = jnp.where(va == e, run[e], pos)
            runa = tuple(run[e] + jnp.where(va == e, 1, 0) for e in range(E))
            posA_v[pl.ds(k * 16, 16)] = pos
            vb = eB_v[pl.ds(k * 16, 16)]
            pos2 = jnp.zeros((16,), _i32)
            for e in range(E):
                pos2 = jnp.where(vb == e, runa[e], pos2)
            runb = tuple(runa[e] + jnp.where(vb == e, 1, 0) for e in range(E))
            posB_v[pl.ds(k * 16, 16)] = pos2
            return runb

        lax.fori_loop(0, L // 16, pbody, run0)

        pltpu.sync_copy(posA_v, posA_hbm)
        pltpu.sync_copy(posB_v, posB_hbm)
        pltpu.async_copy(tid_v, tok_hbm.at[posA_v], sem).wait()
        pltpu.async_copy(tid_v, tok_hbm.at[posB_v], sem).wait()


_GCH = MAX_P // NW // 2


@functools.partial(
    pl.kernel,
    mesh=_mesh,
    out_type=jax.ShapeDtypeStruct((MAX_P, DIM), _f32),
    scratch_types=[
        pltpu.VMEM((_GCH,), _i32),
        pltpu.VMEM((_GCH, DIM), _f32),
        pltpu.SemaphoreType.DMA,
    ],
)
def sc_gather(h2_hbm, tok_hbm, xs_hbm, idx_v, rows_v, sem):
    wid = _widx()
    base = wid * (MAX_P // NW)
    for c in range(2):
        b = base + c * _GCH
        pltpu.sync_copy(tok_hbm.at[pl.ds(b, _GCH)], idx_v)
        pltpu.async_copy(h2_hbm.at[idx_v], rows_v, sem).wait()
        pltpu.sync_copy(rows_v, xs_hbm.at[pl.ds(b, _GCH)])


_CCH = L // NW // 2


@functools.partial(
    pl.kernel,
    mesh=_mesh,
    out_type=jax.ShapeDtypeStruct((L, DIM), _f32),
    scratch_types=[
        pltpu.VMEM((_CCH,), _i32),
        pltpu.VMEM((_CCH,), _i32),
        pltpu.VMEM((_CCH, 16), _f32),
        pltpu.VMEM((_CCH, 16), _f32),
        pltpu.VMEM((_CCH, DIM), _f32),
        pltpu.VMEM((_CCH, DIM), _f32),
        pltpu.VMEM((_CCH, DIM), _f32),
        pltpu.SemaphoreType.DMA,
    ],
)
def sc_combine(ys_hbm, posA_hbm, posB_hbm, wA_hbm, wB_hbm, x2_hbm, out_hbm,
               pA_v, pB_v, wa_v, wb_v, yA_v, yB_v, o_v, sem):
    wid = _widx()
    for c in range(2):
        b = wid * (L // NW) + c * _CCH
        pltpu.sync_copy(posA_hbm.at[pl.ds(b, _CCH)], pA_v)
        pltpu.sync_copy(posB_hbm.at[pl.ds(b, _CCH)], pB_v)
        pltpu.sync_copy(wA_hbm.at[pl.ds(b, _CCH)], wa_v)
        pltpu.sync_copy(wB_hbm.at[pl.ds(b, _CCH)], wb_v)
        pltpu.async_copy(ys_hbm.at[pA_v], yA_v, sem).wait()
        pltpu.async_copy(ys_hbm.at[pB_v], yB_v, sem).wait()
        pltpu.sync_copy(x2_hbm.at[pl.ds(b, _CCH)], o_v)

        def rbody(r, carry):
            wa = wa_v[r, :]
            wb = wb_v[r, :]
            for k in range(DIM // 16):
                sl = pl.ds(k * 16, 16)
                o_v[r, sl] = (o_v[r, sl] + wa * yA_v[r, sl]
                              + wb * yB_v[r, sl])
            return carry
        lax.fori_loop(0, _CCH, rbody, 0)
        pltpu.sync_copy(o_v, out_hbm.at[pl.ds(b, _CCH)])


def moe_routed(h2, eA, eB, wA16, wB16, x2, w1_b, b1, w2_b, b2):
    cnt = sc_hist(eA, eB)
    base, blkm = tc_meta(cnt.reshape(E, 16))
    posA, posB, tok = sc_pos(eA, eB, base.reshape(-1))
    xs = sc_gather(h2, tok)

    def _ker_body(meta_ref, xs_ref, w1_ref, b1_ref, w2_ref, b2_ref, ys_ref):
        b = pl.program_id(0)

        @pl.when(b < meta_ref[32])
        def _():
            xb = xs_ref[...].astype(_bf16)
            m1 = jnp.dot(xb, w1_ref[0], preferred_element_type=_f32) + b1_ref[0]
            a = jax.nn.gelu(m1)
            ys_ref[...] = (jnp.dot(a.astype(_bf16), w2_ref[0],
                                   preferred_element_type=_f32) + b2_ref[0])

    ys = pl.pallas_call(
        _ker_body,
        grid_spec=pltpu.PrefetchScalarGridSpec(
            num_scalar_prefetch=1,
            grid=(NBLK_MOE,),
            in_specs=[
                pl.BlockSpec((SB, DIM), lambda b, m: (b, 0)),
                pl.BlockSpec((1, DIM, 4 * DIM), lambda b, m: (m[b], 0, 0)),
                pl.BlockSpec((1, 1, 4 * DIM), lambda b, m: (m[b], 0, 0)),
                pl.BlockSpec((1, 4 * DIM, DIM), lambda b, m: (m[b], 0, 0)),
                pl.BlockSpec((1, 1, DIM), lambda b, m: (m[b], 0, 0)),
            ],
            out_specs=pl.BlockSpec((SB, DIM), lambda b, m: (b, 0)),
        ),
        out_shape=jax.ShapeDtypeStruct((MAX_P, DIM), _f32),
    )(blkm.reshape(-1), xs, w1_b, b1[:, None, :], w2_b, b2[:, None, :])

    return sc_combine(ys, posA, posB, wA16, wB16, x2)


def _rmsn(v):
    return v * jax.lax.rsqrt(jnp.mean(v * v, axis=-1, keepdims=True) + 1e-8)


def _silu(v):
    return v * jax.nn.sigmoid(v)


def _km_body(x_ref, win_ref, wc_ref, cb_ref, wx_ref, wdt_ref, bdt_ref,
             dp_ref, alogt_ref, wout_ref, wg_ref,
             h2_ref, ea_ref, eb_ref, wa_ref, wb_ref,
             halo_ref, h_ref, dts_ref, us_ref, bs_ref, cs_ref, ys_ref):
    i = pl.program_id(0)
    xb = x_ref[...]
    h1 = _rmsn(xb)
    xz = jnp.dot(h1.astype(_bf16), win_ref[...], preferred_element_type=_f32)
    xi_raw = xz[:, :D_INNER]
    z = xz[:, D_INNER:]

    halo = jnp.where(i > 0, halo_ref[...], 0.0)
    ext8 = jnp.concatenate([halo, xi_raw], axis=0).astype(_bf16)
    halo_ref[...] = xi_raw[SB - 8:]
    wc = wc_ref[...]
    rs = jax.lax.broadcasted_iota(jnp.int32, (SB, SB + 8), 0)
    cs = jax.lax.broadcasted_iota(jnp.int32, (SB, SB + 8), 1)
    xc = cb_ref[...] * jnp.ones((SB, 1), _f32)
    for j in range(D_CONV):
        shj = (cs == rs + 5 + j).astype(_bf16)
        xc = xc + wc[j:j + 1] * jax.lax.dot_general(
            shj, ext8, (((1,), (0,)), ((), ())), preferred_element_type=_f32)
    xi = _silu(xc)
    x_dbl = jnp.dot(xi, wx_ref[...], preferred_element_type=_f32)
    dt = jax.nn.softplus(
        jnp.dot(x_dbl[:, :DT_RANK], wdt_ref[...], preferred_element_type=_f32)
        + bdt_ref[...])
    dts_ref[...] = dt
    us_ref[...] = dt * xi
    bs_ref[...] = x_dbl[:, DT_RANK:DT_RANK + D_STATE]
    cs_ref[...] = x_dbl[:, DT_RANK + D_STATE:DT_RANK + 2 * D_STATE]

    @pl.when(i == 0)
    def _():
        h_ref[...] = jnp.zeros_like(h_ref)

    at = -jnp.exp(alogt_ref[...])
    at_tile = jnp.concatenate([at] * KS, axis=0)
    r1 = jax.lax.broadcasted_iota(jnp.int32, (KS * D_STATE, KS), 0)
    c1 = jax.lax.broadcasted_iota(jnp.int32, (KS * D_STATE, KS), 1)
    rsel = (r1 // D_STATE == c1).astype(_bf16)
    r2 = jax.lax.broadcasted_iota(jnp.int32, (KS * D_STATE, D_STATE), 0)
    c2 = jax.lax.broadcasted_iota(jnp.int32, (KS * D_STATE, D_STATE), 1)
    nmask = (r2 % D_STATE == c2).astype(_f32)
    ones16 = jnp.ones((D_STATE, 1), _bf16)
    r3 = jax.lax.broadcasted_iota(jnp.int32, (D_STATE, KS * D_STATE), 0)
    c3 = jax.lax.broadcasted_iota(jnp.int32, (D_STATE, KS * D_STATE), 1)
    tile16 = (c3 % D_STATE == r3).astype(_bf16)
    r4 = jax.lax.broadcasted_iota(jnp.int32, (KS, KS * D_STATE), 0)
    c4 = jax.lax.broadcasted_iota(jnp.int32, (KS, KS * D_STATE), 1)
    smask = (c4 // D_STATE == r4).astype(_f32)

    def outer(g, h):
        s = g * KS
        dt_blk = dts_ref[pl.ds(s, KS), :]
        u_blk = us_ref[pl.ds(s, KS), :]
        b_blk = bs_ref[pl.ds(s, KS), :]
        c_blk = cs_ref[pl.ds(s, KS), :]
        dtrep = jax.lax.dot_general(rsel, dt_blk.astype(_bf16),
                                    (((1,), (0,)), ((), ())),
                                    preferred_element_type=_f32)
        da = jnp.exp(dtrep * at_tile)
        urep = jax.lax.dot_general(rsel, u_blk.astype(_bf16),
                                   (((1,), (0,)), ((), ())),
                                   preferred_element_type=_f32)
        brep = jax.lax.dot_general(rsel, b_blk.astype(_bf16),
                                   (((1,), (0,)), ((), ())),
                                   preferred_element_type=_f32)
        bcol = jnp.dot((brep * nmask).astype(_bf16), ones16,
                       preferred_element_type=_f32)
        dbx = bcol * urep
        ctile = jnp.dot(c_blk.astype(_bf16), tile16,
                        preferred_element_type=_f32)
        sm = (ctile * smask).astype(_bf16)
        hs = []
        for t in range(KS):
            h = (h * da[t * D_STATE:(t + 1) * D_STATE]
                 + dbx[t * D_STATE:(t + 1) * D_STATE])
            hs.append(h)
        hstk = jnp.concatenate(hs, axis=0).astype(_bf16)
        y_blk = jnp.dot(sm, hstk, preferred_element_type=_f32)
        ys_ref[pl.ds(s, KS), :] = y_blk
        return h

    h = jax.lax.fori_loop(0, SB // KS, outer, h_ref[...])
    h_ref[...] = h

    yg = (ys_ref[...] + xi * dp_ref[...]) * _silu(z)
    y2 = jnp.dot(yg.astype(_bf16), wout_ref[...], preferred_element_type=_f32)
    h2 = _rmsn(y2 + h1)
    h2_ref[...] = h2
    logits = jnp.dot(h2, wg_ref[...], preferred_element_type=_f32)
    ii = jax.lax.broadcasted_iota(jnp.int32, (SB, E), 1)
    v1 = jnp.max(logits, axis=1, keepdims=True)
    i1 = jnp.min(jnp.where(logits == v1, ii, E), axis=1, keepdims=True)
    l2 = jnp.where(ii == i1, -1e30, logits)
    v2 = jnp.max(l2, axis=1, keepdims=True)
    i2 = jnp.min(jnp.where(l2 == v2, ii, E), axis=1, keepdims=True)
    w1 = jax.nn.sigmoid(v1 - v2)
    ea_ref[...] = i1
    eb_ref[...] = i2
    wa_ref[...] = w1 * jnp.ones((SB, 16), _f32)
    wb_ref[...] = (1.0 - w1) * jnp.ones((SB, 16), _f32)


def kernel(x, W_in, conv_w, conv_b, W_xproj, W_dt, b_dt, A_log, Dp, W_out,
           W_gate, W1, b1, W2, b2):
    x2 = x[0]
    win_b = W_in.astype(_bf16)
    wout_b = W_out.astype(_bf16)
    w1_b = W1.astype(_bf16)
    w2_b = W2.astype(_bf16)
    wc = jnp.transpose(conv_w[:, 0, :], (1, 0))
    alogt = jnp.transpose(A_log, (1, 0))

    h2f, eac, ebc, wa16, wb16 = pl.pallas_call(
        _km_body,
        grid=(NSB,),
        in_specs=[
            pl.BlockSpec((SB, DIM), lambda i: (i, 0)),
            pl.BlockSpec((DIM, 2 * D_INNER), lambda i: (0, 0)),
            pl.BlockSpec((4, D_INNER), lambda i: (0, 0)),
            pl.BlockSpec((1, D_INNER), lambda i: (0, 0)),
            pl.BlockSpec((D_INNER, DT_RANK + 2 * D_STATE), lambda i: (0, 0)),
            pl.BlockSpec((DT_RANK, D_INNER), lambda i: (0, 0)),
            pl.BlockSpec((1, D_INNER), lambda i: (0, 0)),
            pl.BlockSpec((1, D_INNER), lambda i: (0, 0)),
            pl.BlockSpec((D_STATE, D_INNER), lambda i: (0, 0)),
            pl.BlockSpec((D_INNER, DIM), lambda i: (0, 0)),
            pl.BlockSpec((DIM, E), lambda i: (0, 0)),
        ],
        out_specs=[
            pl.BlockSpec((SB, DIM), lambda i: (i, 0)),
            pl.BlockSpec((SB, 1), lambda i: (i, 0)),
            pl.BlockSpec((SB, 1), lambda i: (i, 0)),
            pl.BlockSpec((SB, 16), lambda i: (i, 0)),
            pl.BlockSpec((SB, 16), lambda i: (i, 0)),
        ],
        out_shape=[
            jax.ShapeDtypeStruct((L, DIM), _f32),
            jax.ShapeDtypeStruct((L, 1), jnp.int32),
            jax.ShapeDtypeStruct((L, 1), jnp.int32),
            jax.ShapeDtypeStruct((L, 16), _f32),
            jax.ShapeDtypeStruct((L, 16), _f32),
        ],
        scratch_shapes=[
            pltpu.VMEM((8, D_INNER), _f32),
            pltpu.VMEM((D_STATE, D_INNER), _f32),
            pltpu.VMEM((SB, D_INNER), _f32),
            pltpu.VMEM((SB, D_INNER), _f32),
            pltpu.VMEM((SB, D_STATE), _f32),
            pltpu.VMEM((SB, D_STATE), _f32),
            pltpu.VMEM((SB, D_INNER), _f32),
        ],
    )(x2, win_b, wc, conv_b[None, :], W_xproj, W_dt, b_dt[None, :],
      Dp[None, :], alogt, wout_b, W_gate)

    out = moe_routed(h2f, eac.reshape(L), ebc.reshape(L), wa16, wb16, x2,
                     w1_b, b1, w2_b, b2)

    return out[None]
